# baseline (speedup 1.0000x reference)
import functools
import jax
import jax.numpy as jnp
from jax import lax
from jax.experimental import pallas as pl
from jax.experimental.pallas import tpu as pltpu
from jax.experimental.pallas import tpu_sc as plsc

FREQ_CONST = 10000.0
NUM_CORES = 2
NUM_SUBCORES = 16
NUM_WORKERS = NUM_CORES * NUM_SUBCORES
LANES = 16
CS_W = 128


def _rope_sc(table, idx, cs_t, *, S, D, rows_per_w, seqs_per_w):
  n_rows = idx.shape[0]
  mesh = plsc.VectorSubcoreMesh(
      core_axis_name="c", subcore_axis_name="s",
      num_cores=NUM_CORES, num_subcores=NUM_SUBCORES)
  n_blk = D // 32

  NBUF = 3
  PREF = 2

  @functools.partial(
      pl.kernel,
      out_type=jax.ShapeDtypeStruct((n_rows, D), jnp.float32),
      mesh=mesh,
      compiler_params=pltpu.CompilerParams(use_tc_tiling_on_sc=True, needs_layout_passes=False),
      scratch_types=dict(
          bufs=(pltpu.VMEM((S * D // 2,), jnp.int32),) * NBUF,
          obufs=(pltpu.VMEM((S, D), jnp.float32),) * NBUF,
          idx_s=(pltpu.VMEM((S,), jnp.int32),) * NBUF,
          cs_v=pltpu.VMEM((S, CS_W), jnp.float32),
          gsems=(pltpu.SemaphoreType.DMA,) * NBUF,
          osems=(pltpu.SemaphoreType.DMA,) * NBUF,
      ),
  )
  def run(table_hbm, idx_hbm, cs_hbm, out_hbm, bufs, obufs, idx_s, cs_v,
          gsems, osems):
    wid = lax.axis_index("s") * NUM_CORES + lax.axis_index("c")
    base = wid * rows_per_w
    pltpu.sync_copy(cs_hbm, cs_v)

    def fetch_rows(g, b):
      pltpu.sync_copy(idx_hbm.at[pl.ds(base + g * S, S)], idx_s[b])

      def issue(r0, j_lo):
        vec = idx_s[b][pl.ds(r0, LANES)] * (D // 2)
        for j in range(j_lo, LANES):
          pltpu.make_async_copy(
              table_hbm.at[pl.ds(pl.multiple_of(vec[j], 8), D // 2)],
              bufs[b].at[pl.ds((r0 + j) * (D // 2), D // 2)], gsems[b]).start()

      n_full = S // LANES

      @pl.loop(0, n_full * LANES, step=LANES)
      def _blk(r0):
        issue(r0, 0)

      if S % LANES:
        issue(S - LANES, LANES - S % LANES)

    def fetch_drain(b):
      pltpu.make_async_copy(
          table_hbm.at[pl.ds(0, S * D // 2)], bufs[b], gsems[b]).wait()

    def out_desc(g, b):
      return pltpu.make_async_copy(
          obufs[b], out_hbm.at[pl.ds(base + g * S, S)], osems[b])

    lane = lax.iota(jnp.int32, LANES)
    rot1 = (lane + 1) & (LANES - 1)
    zero_idx = jnp.zeros((LANES,), jnp.int32)
    last_lane = lane == (LANES - 1)
    odd_lane = (lane & 1) == 1
    lo_idx = lane >> 1
    hi_idx = (LANES // 2) + (lane >> 1)

    def vgather(v, idx):
      dnums = lax.GatherDimensionNumbers(
          offset_dims=(), collapsed_slice_dims=(0,), start_index_map=(0,))
      return lax.gather(v, idx[:, None], dnums, (1,),
                        mode=lax.GatherScatterMode.PROMISE_IN_BOUNDS)

    def compute(buf, obuf):
      @pl.loop(0, S, unroll=4)
      def _row(s):
        a, bb = [], []
        for k in range(n_blk):
          w = buf[pl.ds(s * (D // 2) + LANES * k, LANES)]
          a.append(plsc.bitcast(w << 16, jnp.float32))
          bb.append(plsc.bitcast(w & jnp.int32(-65536), jnp.float32))
        for k in range(n_blk):
          an = a[(k + 1) % n_blk]
          sh = jnp.where(last_lane, vgather(an, zero_idx), vgather(a[k], rot1))
          ce = cs_v[s, pl.ds(16 * k, LANES)]
          co = cs_v[s, pl.ds(32 + 16 * k, LANES)]
          se = cs_v[s, pl.ds(64 + 16 * k, LANES)]
          so = cs_v[s, pl.ds(96 + 16 * k, LANES)]
          oe = a[k] * ce + bb[k] * se
          oo = bb[k] * co + sh * so
          first = jnp.where(odd_lane, vgather(oo, lo_idx), vgather(oe, lo_idx))
          second = jnp.where(odd_lane, vgather(oo, hi_idx), vgather(oe, hi_idx))
          obuf[s, pl.ds(32 * k, LANES)] = first
          obuf[s, pl.ds(32 * k + LANES, LANES)] = second

    for b in range(PREF):
      fetch_rows(b, b)

    @pl.loop(0, seqs_per_w + (-seqs_per_w) % NBUF, step=NBUF)
    def _ring(gg):
      for b in range(NBUF):
        g = gg + b

        @pl.when(g < seqs_per_w)
        def _():
          fetch_drain(b)
          compute(bufs[b], obufs[b])
          out_desc(g, b).start()

          @pl.when(g + PREF < seqs_per_w)
          def _():
            @pl.when(g + PREF >= NBUF)
            def _():
              out_desc(g + PREF - NBUF, (b + PREF) % NBUF).wait()
            fetch_rows(g + PREF, (b + PREF) % NBUF)

    for b in range(NBUF):
      g = seqs_per_w - NBUF + b
      out_desc(g, g % NBUF).wait()

  return run(table, idx, cs_t)


def kernel(ids, token_embedding):
  B, S = ids.shape
  V, D = token_embedding.shape
  n_rows = B * S
  assert n_rows % NUM_WORKERS == 0
  rows_per_w = n_rows // NUM_WORKERS
  assert rows_per_w % S == 0
  seqs_per_w = rows_per_w // S

  ids_flat = ids.reshape(n_rows).astype(jnp.int32)
  table_16 = lax.optimization_barrier(token_embedding.astype(jnp.bfloat16))
  table_b = jax.lax.bitcast_convert_type(table_16.reshape(-1, 2), jnp.int32)
  i = jnp.arange(D, dtype=jnp.float32)
  freq = 1.0 / (FREQ_CONST ** (2.0 * jnp.floor(i / 2.0) / D))
  theta = jnp.arange(S, dtype=jnp.float32)[:, None] * freq[None, :]
  cos_t = jnp.cos(theta)
  sin_t = jnp.sin(theta)
  cs_t = jnp.concatenate(
      [cos_t[:, 0::2], cos_t[:, 1::2], sin_t[:, 0::2], sin_t[:, 1::2]],
      axis=1)

  out = _rope_sc(table_b, ids_flat, cs_t,
                 S=S, D=D, rows_per_w=rows_per_w, seqs_per_w=seqs_per_w)
  return out.reshape(B, S, D)


# speedup vs baseline: 55.2222x; 55.2222x over previous
import functools
import jax
import jax.numpy as jnp
from jax import lax
from jax.experimental import pallas as pl
from jax.experimental.pallas import tpu as pltpu
from jax.experimental.pallas import tpu_sc as plsc

FREQ_CONST = 10000.0
NUM_CORES = 2
NUM_SUBCORES = 16
NUM_WORKERS = NUM_CORES * NUM_SUBCORES
LANES = 16
CS_W = 128


def _rope_sc(table, idx, cs_t, *, S, D, rows_per_w, seqs_per_w):
  n_rows = idx.shape[0]
  HALF_V = table.shape[1]
  mesh = plsc.VectorSubcoreMesh(
      core_axis_name="c", subcore_axis_name="s",
      num_cores=NUM_CORES, num_subcores=NUM_SUBCORES)
  n_chunks = D // LANES

  NBUF = 3
  PREF = 2

  @functools.partial(
      pl.kernel,
      out_type=jax.ShapeDtypeStruct((n_rows, D), jnp.float32),
      mesh=mesh,
      compiler_params=pltpu.CompilerParams(use_tc_tiling_on_sc=True),
      scratch_types=dict(
          bufs=(pltpu.VMEM((S, D), jnp.float32),) * NBUF,
          idx_s=(pltpu.VMEM((S,), jnp.int32),) * NBUF,
          cs_v=pltpu.VMEM((S, CS_W), jnp.float32),
          gsems=(pltpu.SemaphoreType.DMA,) * NBUF,
          osems=(pltpu.SemaphoreType.DMA,) * NBUF,
      ),
  )
  def run(table_hbm, idx_hbm, cs_hbm, out_hbm, bufs, idx_s, cs_v,
          gsems, osems):
    wid = lax.axis_index("s") * NUM_CORES + lax.axis_index("c")
    base = wid * rows_per_w
    pltpu.sync_copy(cs_hbm, cs_v)

    def fetch_rows(g, b):
      pltpu.sync_copy(idx_hbm.at[pl.ds(base + g * S, S)], idx_s[b])

      def issue(r0, j_lo):
        vec = idx_s[b][pl.ds(r0, LANES)]
        for j in range(j_lo, LANES):
          v = vec[j]
          hi = (v >= HALF_V).astype(jnp.int32)
          r = v - hi * HALF_V
          pltpu.make_async_copy(
              table_hbm.at[hi, pl.ds(r, 1)],
              bufs[b].at[pl.ds(r0 + j, 1)], gsems[b]).start()

      n_full = S // LANES

      @pl.loop(0, n_full * LANES, step=LANES)
      def _blk(r0):
        issue(r0, 0)

      if S % LANES:
        issue(S - LANES, LANES - S % LANES)

    def fetch_drain(b):
      pltpu.make_async_copy(
          table_hbm.at[0, pl.ds(0, S)], bufs[b], gsems[b]).wait()

    def out_desc(g, b):
      return pltpu.make_async_copy(
          bufs[b], out_hbm.at[pl.ds(base + g * S, S)], osems[b])

    lane = lax.iota(jnp.int32, LANES)
    rot1 = (lane + 1) & (LANES - 1)
    zero_idx = jnp.zeros((LANES,), jnp.int32)
    last_lane = lane == (LANES - 1)

    def vgather(v, idx):
      dnums = lax.GatherDimensionNumbers(
          offset_dims=(), collapsed_slice_dims=(0,), start_index_map=(0,))
      return lax.gather(v, idx[:, None], dnums, (1,),
                        mode=lax.GatherScatterMode.PROMISE_IN_BOUNDS)

    def compute(buf):
      @pl.loop(0, S, unroll=4)
      def _row(s):
        e = [buf[s, pl.ds(c * LANES, LANES)] for c in range(n_chunks)]
        vals = []
        for c in range(n_chunks):
          nxt = e[(c + 1) % n_chunks]
          esh = jnp.where(last_lane, vgather(nxt, zero_idx), vgather(e[c], rot1))
          co = cs_v[s, pl.ds(c * LANES, LANES)]
          si = cs_v[s, pl.ds(D + c * LANES, LANES)]
          vals.append(e[c] * co + esh * si)
        for c in range(n_chunks):
          buf[s, pl.ds(c * LANES, LANES)] = vals[c]

    for b in range(PREF):
      fetch_rows(b, b)

    @pl.loop(0, seqs_per_w + (-seqs_per_w) % NBUF, step=NBUF)
    def _ring(gg):
      for b in range(NBUF):
        g = gg + b

        @pl.when(g < seqs_per_w)
        def _():
          fetch_drain(b)
          compute(bufs[b])
          out_desc(g, b).start()

          @pl.when(g + PREF < seqs_per_w)
          def _():
            @pl.when(g + PREF >= NBUF)
            def _():
              out_desc(g + PREF - NBUF, (b + PREF) % NBUF).wait()
            fetch_rows(g + PREF, (b + PREF) % NBUF)

    for b in range(NBUF):
      g = seqs_per_w - NBUF + b
      out_desc(g, g % NBUF).wait()

  return run(table, idx, cs_t)


def kernel(ids, token_embedding):
  B, S = ids.shape
  V, D = token_embedding.shape
  n_rows = B * S
  assert n_rows % NUM_WORKERS == 0
  rows_per_w = n_rows // NUM_WORKERS
  assert rows_per_w % S == 0
  seqs_per_w = rows_per_w // S

  ids_flat = ids.reshape(n_rows).astype(jnp.int32)
  i = jnp.arange(D, dtype=jnp.float32)
  freq = 1.0 / (FREQ_CONST ** (2.0 * jnp.floor(i / 2.0) / D))
  theta = jnp.arange(S, dtype=jnp.float32)[:, None] * freq[None, :]
  cs_t = jnp.concatenate([jnp.cos(theta), jnp.sin(theta)], axis=1)

  table_3d = token_embedding.reshape(2, V // 2, D)
  out = _rope_sc(table_3d, ids_flat, cs_t,
                 S=S, D=D, rows_per_w=rows_per_w, seqs_per_w=seqs_per_w)
  return out.reshape(B, S, D)


# speedup vs baseline: 55.3058x; 1.0015x over previous
import functools
import jax
import jax.numpy as jnp
from jax import lax
from jax.experimental import pallas as pl
from jax.experimental.pallas import tpu as pltpu
from jax.experimental.pallas import tpu_sc as plsc

FREQ_CONST = 10000.0
NUM_CORES = 2
NUM_SUBCORES = 16
NUM_WORKERS = NUM_CORES * NUM_SUBCORES
LANES = 16
CS_W = 128


def _rope_sc(table, idx, cs_t, *, S, D, rows_per_w, seqs_per_w):
  n_rows = idx.shape[0]
  HALF_V = table.shape[1]
  mesh = plsc.VectorSubcoreMesh(
      core_axis_name="c", subcore_axis_name="s",
      num_cores=NUM_CORES, num_subcores=NUM_SUBCORES)
  n_chunks = D // LANES

  NBUF = 4
  PREF = 2

  @functools.partial(
      pl.kernel,
      out_type=jax.ShapeDtypeStruct((n_rows, D), jnp.float32),
      mesh=mesh,
      compiler_params=pltpu.CompilerParams(use_tc_tiling_on_sc=True),
      scratch_types=dict(
          bufs=(pltpu.VMEM((S, D), jnp.float32),) * NBUF,
          idx_s=(pltpu.VMEM((S,), jnp.int32),) * NBUF,
          cs_v=pltpu.VMEM((S, CS_W), jnp.float32),
          gsems=(pltpu.SemaphoreType.DMA,) * NBUF,
          osems=(pltpu.SemaphoreType.DMA,) * NBUF,
      ),
  )
  def run(table_hbm, idx_hbm, cs_hbm, out_hbm, bufs, idx_s, cs_v,
          gsems, osems):
    wid = lax.axis_index("s") * NUM_CORES + lax.axis_index("c")
    base = wid * rows_per_w
    pltpu.sync_copy(cs_hbm, cs_v)

    def fetch_rows(g, b):
      pltpu.sync_copy(idx_hbm.at[pl.ds(base + g * S, S)], idx_s[b])

      def issue(r0, j_lo):
        vec = idx_s[b][pl.ds(r0, LANES)]
        for j in range(j_lo, LANES):
          v = vec[j]
          hi = (v >= HALF_V).astype(jnp.int32)
          r = v - hi * HALF_V
          pltpu.make_async_copy(
              table_hbm.at[hi, pl.ds(r, 1)],
              bufs[b].at[pl.ds(r0 + j, 1)], gsems[b]).start()

      n_full = S // LANES

      @pl.loop(0, n_full * LANES, step=LANES)
      def _blk(r0):
        issue(r0, 0)

      if S % LANES:
        issue(S - LANES, LANES - S % LANES)

    def fetch_drain(b):
      pltpu.make_async_copy(
          table_hbm.at[0, pl.ds(0, S)], bufs[b], gsems[b]).wait()

    def out_desc(g, b):
      return pltpu.make_async_copy(
          bufs[b], out_hbm.at[pl.ds(base + g * S, S)], osems[b])

    lane = lax.iota(jnp.int32, LANES)
    rot1 = (lane + 1) & (LANES - 1)
    zero_idx = jnp.zeros((LANES,), jnp.int32)
    last_lane = lane == (LANES - 1)

    def vgather(v, idx):
      dnums = lax.GatherDimensionNumbers(
          offset_dims=(), collapsed_slice_dims=(0,), start_index_map=(0,))
      return lax.gather(v, idx[:, None], dnums, (1,),
                        mode=lax.GatherScatterMode.PROMISE_IN_BOUNDS)

    def compute(buf):
      @pl.loop(0, S, unroll=8)
      def _row(s):
        e = [buf[s, pl.ds(c * LANES, LANES)] for c in range(n_chunks)]
        vals = []
        for c in range(n_chunks):
          nxt = e[(c + 1) % n_chunks]
          esh = jnp.where(last_lane, vgather(nxt, zero_idx), vgather(e[c], rot1))
          co = cs_v[s, pl.ds(c * LANES, LANES)]
          si = cs_v[s, pl.ds(D + c * LANES, LANES)]
          vals.append(e[c] * co + esh * si)
        for c in range(n_chunks):
          buf[s, pl.ds(c * LANES, LANES)] = vals[c]

    for b in range(PREF):
      fetch_rows(b, b)

    @pl.loop(0, seqs_per_w + (-seqs_per_w) % NBUF, step=NBUF)
    def _ring(gg):
      for b in range(NBUF):
        g = gg + b

        @pl.when(g < seqs_per_w)
        def _():
          fetch_drain(b)
          compute(bufs[b])
          out_desc(g, b).start()

          @pl.when(g + PREF < seqs_per_w)
          def _():
            @pl.when(g + PREF >= NBUF)
            def _():
              out_desc(g + PREF - NBUF, (b + PREF) % NBUF).wait()
            fetch_rows(g + PREF, (b + PREF) % NBUF)

    for b in range(NBUF):
      g = seqs_per_w - NBUF + b
      out_desc(g, g % NBUF).wait()

  return run(table, idx, cs_t)


def kernel(ids, token_embedding):
  B, S = ids.shape
  V, D = token_embedding.shape
  n_rows = B * S
  assert n_rows % NUM_WORKERS == 0
  rows_per_w = n_rows // NUM_WORKERS
  assert rows_per_w % S == 0
  seqs_per_w = rows_per_w // S

  ids_flat = ids.reshape(n_rows).astype(jnp.int32)
  i = jnp.arange(D, dtype=jnp.float32)
  freq = 1.0 / (FREQ_CONST ** (2.0 * jnp.floor(i / 2.0) / D))
  theta = jnp.arange(S, dtype=jnp.float32)[:, None] * freq[None, :]
  cs_t = jnp.concatenate([jnp.cos(theta), jnp.sin(theta)], axis=1)

  table_3d = token_embedding.reshape(2, V // 2, D)
  out = _rope_sc(table_3d, ids_flat, cs_t,
                 S=S, D=D, rows_per_w=rows_per_w, seqs_per_w=seqs_per_w)
  return out.reshape(B, S, D)
